# baseline (speedup 1.0000x reference)
import functools

import jax
import jax.numpy as jnp
from jax.experimental import pallas as pl
from jax.experimental.pallas import tpu as pltpu

TAU = 0.07
TOPK = 10


def _topk_iteration(k, simT_ref, buf, iota_ref, aux_ref, knn_ref, qb,
                    p_total):
    x = simT_ref[buf]
    iota_p = iota_ref[...]
    m = jnp.max(x, axis=0, keepdims=True)
    cand = jnp.where(x == m, iota_p, float(p_total))
    i = jnp.min(cand, axis=0, keepdims=True)
    if k == 0:
        aux_ref[0:1, :] = m
    else:
        aux_ref[0:1, :] += m
    knn_ref[k:k + 1, :] = i.astype(jnp.int32)
    if k < TOPK - 1:
        simT_ref[buf] = jnp.where(iota_p == i, -jnp.inf, x)
    else:
        aux_ref[0:1, :] = aux_ref[0:1, :] / float(TOPK)


def _snn_kernel(hb_ref, ptb_ref, lb_ref, pi_ref, aux_ref, knn_ref,
                acc_ref, simT_ref, iota_ref, *, n_q, n_p, qb, pb, c_real,
                p_total):
    iq = pl.program_id(0)
    ip = pl.program_id(1)

    @pl.when(jnp.logical_and(iq == 0, ip == 0))
    def _init_iota():
        iota_ref[...] = jax.lax.broadcasted_iota(
            jnp.int32, (p_total, qb), 0).astype(jnp.float32)

    @pl.when(iq < n_q)
    def _compute():
        sim = jnp.dot(hb_ref[...], ptb_ref[...],
                      preferred_element_type=jnp.float32)
        simT_ref[iq % 2, pl.ds(ip * pb, pb), :] = sim.T

        e = jnp.exp(sim * (1.0 / TAU))
        contrib = jnp.dot(e.astype(jnp.bfloat16), lb_ref[...],
                          preferred_element_type=jnp.float32)

        @pl.when(ip == 0)
        def _init():
            acc_ref[...] = contrib

        @pl.when(ip > 0)
        def _accum():
            acc_ref[...] += contrib

        @pl.when(ip == n_p - 1)
        def _finalize_pi():
            acc = acc_ref[...]
            cpad = acc.shape[1]
            iota_c = jax.lax.broadcasted_iota(jnp.int32, (qb, cpad), 1)
            den = jnp.sum(jnp.where(iota_c == c_real, acc, 0.0), axis=1,
                          keepdims=True)
            pi_ref[...] = acc / den

    per_step = -(-TOPK // n_p)
    prev_buf = (iq + 1) % 2

    @pl.when(iq > 0)
    def _topk():
        for c in range((TOPK + per_step - 1) // per_step):
            ks = [k for k in range(c * per_step, min((c + 1) * per_step, TOPK))]

            @pl.when(ip == c)
            def _run(ks=ks):
                for k in ks:
                    _topk_iteration(k, simT_ref, prev_buf, iota_ref,
                                    aux_ref, knn_ref, qb, p_total)


def kernel(h, peaks, labels):
    q, d = h.shape
    p_total, c = labels.shape[0], labels.shape[1]

    h_n = h / jnp.linalg.norm(h, axis=-1, keepdims=True)
    p_n = peaks / jnp.linalg.norm(peaks, axis=-1, keepdims=True)

    hb = h_n.astype(jnp.bfloat16)
    ptb = p_n.astype(jnp.bfloat16).T

    cpad = ((c + 1 + 127) // 128) * 128
    lb = jnp.pad(labels.astype(jnp.bfloat16), ((0, 0), (0, cpad - c)))
    ones_col = (jax.lax.broadcasted_iota(jnp.int32, (1, cpad), 1) == c)
    lb = jnp.where(ones_col, jnp.bfloat16(1.0), lb)

    qb = 256 if q % 256 == 0 else q
    pb = 1024 if p_total % 1024 == 0 else p_total
    n_q, n_p = q // qb, p_total // pb

    body = functools.partial(_snn_kernel, n_q=n_q, n_p=n_p, qb=qb, pb=pb,
                             c_real=c, p_total=p_total)
    last_q = n_q - 1
    pi_pad, aux, knn_t = pl.pallas_call(
        body,
        grid=(n_q + 1, n_p),
        in_specs=[
            pl.BlockSpec((qb, d), lambda iq, ip: (jnp.minimum(iq, last_q), 0)),
            pl.BlockSpec((d, pb), lambda iq, ip: (0, ip)),
            pl.BlockSpec((pb, cpad), lambda iq, ip: (ip, 0)),
        ],
        out_specs=[
            pl.BlockSpec((qb, cpad),
                         lambda iq, ip: (jnp.minimum(iq, last_q), 0)),
            pl.BlockSpec((8, qb), lambda iq, ip: (0, jnp.maximum(iq - 1, 0))),
            pl.BlockSpec((16, qb), lambda iq, ip: (0, jnp.maximum(iq - 1, 0))),
        ],
        out_shape=[
            jax.ShapeDtypeStruct((q, cpad), jnp.float32),
            jax.ShapeDtypeStruct((8, q), jnp.float32),
            jax.ShapeDtypeStruct((16, q), jnp.int32),
        ],
        scratch_shapes=[
            pltpu.VMEM((qb, cpad), jnp.float32),
            pltpu.VMEM((2, p_total, qb), jnp.float32),
            pltpu.VMEM((p_total, qb), jnp.float32),
        ],
    )(hb, ptb, lb)

    p_i = pi_pad[:, :c]
    density = aux[0]
    knn_indices = knn_t[:TOPK].T
    return p_i, density, knn_indices


# speedup vs baseline: 1.0543x; 1.0543x over previous
import functools

import jax
import jax.numpy as jnp
from jax.experimental import pallas as pl
from jax.experimental.pallas import tpu as pltpu

TAU = 0.07
TOPK = 10


def _topk_iteration(k, simT_ref, buf, iota_ref, aux_ref, knn_ref, qb,
                    p_total):
    x = simT_ref[buf]
    iota_p = iota_ref[...]
    if k == 0:
        m = jnp.max(x, axis=0, keepdims=True)
    else:
        m = aux_ref[1:2, :]
    cand = jnp.where(x == m, iota_p, float(p_total))
    i = jnp.min(cand, axis=0, keepdims=True)
    if k == 0:
        aux_ref[0:1, :] = m
    else:
        aux_ref[0:1, :] += m
    knn_ref[k:k + 1, :] = i.astype(jnp.int32)
    if k < TOPK - 1:
        masked = jnp.where(iota_p == i, -jnp.inf, x)
        simT_ref[buf] = masked
        aux_ref[1:2, :] = jnp.max(masked, axis=0, keepdims=True)
    else:
        aux_ref[0:1, :] = aux_ref[0:1, :] / float(TOPK)


def _snn_kernel(hb_ref, ptb_ref, lb_ref, pi_ref, aux_ref, knn_ref,
                acc_ref, simT_ref, iota_ref, *, n_q, n_p, qb, pb, c_real,
                p_total):
    iq = pl.program_id(0)
    ip = pl.program_id(1)

    @pl.when(jnp.logical_and(iq == 0, ip == 0))
    def _init_iota():
        iota_ref[...] = jax.lax.broadcasted_iota(
            jnp.int32, (p_total, qb), 0).astype(jnp.float32)

    @pl.when(iq < n_q)
    def _compute():
        sim = jnp.dot(hb_ref[...], ptb_ref[...],
                      preferred_element_type=jnp.float32)
        simT_ref[iq % 2, pl.ds(ip * pb, pb), :] = sim.T

        e = jnp.exp(sim * (1.0 / TAU))
        contrib = jnp.dot(e.astype(jnp.bfloat16), lb_ref[...],
                          preferred_element_type=jnp.float32)

        @pl.when(ip == 0)
        def _init():
            acc_ref[...] = contrib

        @pl.when(ip > 0)
        def _accum():
            acc_ref[...] += contrib

        @pl.when(ip == n_p - 1)
        def _finalize_pi():
            acc = acc_ref[...]
            cpad = acc.shape[1]
            iota_c = jax.lax.broadcasted_iota(jnp.int32, (qb, cpad), 1)
            den = jnp.sum(jnp.where(iota_c == c_real, acc, 0.0), axis=1,
                          keepdims=True)
            pi_ref[...] = acc / den

    per_step = -(-TOPK // n_p)
    prev_buf = (iq + 1) % 2

    @pl.when(iq > 0)
    def _topk():
        for c in range((TOPK + per_step - 1) // per_step):
            ks = [k for k in range(c * per_step, min((c + 1) * per_step, TOPK))]

            @pl.when(ip == c)
            def _run(ks=ks):
                for k in ks:
                    _topk_iteration(k, simT_ref, prev_buf, iota_ref,
                                    aux_ref, knn_ref, qb, p_total)


def kernel(h, peaks, labels):
    q, d = h.shape
    p_total, c = labels.shape[0], labels.shape[1]

    h_n = h / jnp.linalg.norm(h, axis=-1, keepdims=True)
    p_n = peaks / jnp.linalg.norm(peaks, axis=-1, keepdims=True)

    hb = h_n.astype(jnp.bfloat16)
    ptb = p_n.astype(jnp.bfloat16).T

    cpad = ((c + 1 + 127) // 128) * 128
    lb = jnp.pad(labels.astype(jnp.bfloat16), ((0, 0), (0, cpad - c)))
    ones_col = (jax.lax.broadcasted_iota(jnp.int32, (1, cpad), 1) == c)
    lb = jnp.where(ones_col, jnp.bfloat16(1.0), lb)

    qb = 256 if q % 256 == 0 else q
    pb = 1024 if p_total % 1024 == 0 else p_total
    n_q, n_p = q // qb, p_total // pb

    body = functools.partial(_snn_kernel, n_q=n_q, n_p=n_p, qb=qb, pb=pb,
                             c_real=c, p_total=p_total)
    last_q = n_q - 1
    pi_pad, aux, knn_t = pl.pallas_call(
        body,
        grid=(n_q + 1, n_p),
        in_specs=[
            pl.BlockSpec((qb, d), lambda iq, ip: (jnp.minimum(iq, last_q), 0)),
            pl.BlockSpec((d, pb), lambda iq, ip: (0, ip)),
            pl.BlockSpec((pb, cpad), lambda iq, ip: (ip, 0)),
        ],
        out_specs=[
            pl.BlockSpec((qb, cpad),
                         lambda iq, ip: (jnp.minimum(iq, last_q), 0)),
            pl.BlockSpec((8, qb), lambda iq, ip: (0, jnp.maximum(iq - 1, 0))),
            pl.BlockSpec((16, qb), lambda iq, ip: (0, jnp.maximum(iq - 1, 0))),
        ],
        out_shape=[
            jax.ShapeDtypeStruct((q, cpad), jnp.float32),
            jax.ShapeDtypeStruct((8, q), jnp.float32),
            jax.ShapeDtypeStruct((16, q), jnp.int32),
        ],
        scratch_shapes=[
            pltpu.VMEM((qb, cpad), jnp.float32),
            pltpu.VMEM((2, p_total, qb), jnp.float32),
            pltpu.VMEM((p_total, qb), jnp.float32),
        ],
    )(hb, ptb, lb)

    p_i = pi_pad[:, :c]
    density = aux[0]
    knn_indices = knn_t[:TOPK].T
    return p_i, density, knn_indices


# speedup vs baseline: 1.0598x; 1.0053x over previous
import functools

import jax
import jax.numpy as jnp
from jax.experimental import pallas as pl
from jax.experimental.pallas import tpu as pltpu

TAU = 0.07
TOPK = 10


def _topk_iteration(k, simT_ref, buf, iota_ref, aux_ref, knn_ref, qb,
                    p_total):
    x = simT_ref[buf]
    iota_p = jax.lax.broadcasted_iota(jnp.int32, x.shape, 0).astype(
        jnp.float32)
    if k == 0:
        m = jnp.max(x, axis=0, keepdims=True)
    else:
        m = aux_ref[1:2, :]
    cand = jnp.where(x == m, iota_p, float(p_total))
    i = jnp.min(cand, axis=0, keepdims=True)
    if k == 0:
        aux_ref[0:1, :] = m
    else:
        aux_ref[0:1, :] += m
    knn_ref[k:k + 1, :] = i.astype(jnp.int32)
    if k < TOPK - 1:
        masked = jnp.where(iota_p == i, -jnp.inf, x)
        simT_ref[buf] = masked
        aux_ref[1:2, :] = jnp.max(masked, axis=0, keepdims=True)
    else:
        aux_ref[0:1, :] = aux_ref[0:1, :] / float(TOPK)


def _snn_kernel(hb_ref, ptb_ref, lb_ref, pi_ref, aux_ref, knn_ref,
                acc_ref, simT_ref, iota_ref, *, n_q, n_p, qb, pb, c_real,
                p_total):
    iq = pl.program_id(0)
    ip = pl.program_id(1)

    @pl.when(jnp.logical_and(iq == 0, ip == 0))
    def _init_iota():
        iota_ref[...] = jax.lax.broadcasted_iota(
            jnp.int32, (p_total, qb), 0).astype(jnp.float32)

    @pl.when(iq < n_q)
    def _compute():
        sim = jnp.dot(hb_ref[...], ptb_ref[...],
                      preferred_element_type=jnp.float32)
        simT_ref[iq % 2, pl.ds(ip * pb, pb), :] = sim.T

        e = jnp.exp(sim * (1.0 / TAU))
        contrib = jnp.dot(e.astype(jnp.bfloat16), lb_ref[...],
                          preferred_element_type=jnp.float32)

        @pl.when(ip == 0)
        def _init():
            acc_ref[...] = contrib

        @pl.when(ip > 0)
        def _accum():
            acc_ref[...] += contrib

        @pl.when(ip == n_p - 1)
        def _finalize_pi():
            acc = acc_ref[...]
            cpad = acc.shape[1]
            iota_c = jax.lax.broadcasted_iota(jnp.int32, (qb, cpad), 1)
            den = jnp.sum(jnp.where(iota_c == c_real, acc, 0.0), axis=1,
                          keepdims=True)
            pi_ref[...] = acc / den

    per_step = -(-TOPK // n_p)
    prev_buf = (iq + 1) % 2

    @pl.when(iq > 0)
    def _topk():
        for c in range((TOPK + per_step - 1) // per_step):
            ks = [k for k in range(c * per_step, min((c + 1) * per_step, TOPK))]

            @pl.when(ip == c)
            def _run(ks=ks):
                for k in ks:
                    _topk_iteration(k, simT_ref, prev_buf, iota_ref,
                                    aux_ref, knn_ref, qb, p_total)


def kernel(h, peaks, labels):
    q, d = h.shape
    p_total, c = labels.shape[0], labels.shape[1]

    h_n = h / jnp.linalg.norm(h, axis=-1, keepdims=True)
    p_n = peaks / jnp.linalg.norm(peaks, axis=-1, keepdims=True)

    hb = h_n.astype(jnp.bfloat16)
    ptb = p_n.astype(jnp.bfloat16).T

    cpad = ((c + 1 + 127) // 128) * 128
    lb = jnp.pad(labels.astype(jnp.bfloat16), ((0, 0), (0, cpad - c)))
    ones_col = (jax.lax.broadcasted_iota(jnp.int32, (1, cpad), 1) == c)
    lb = jnp.where(ones_col, jnp.bfloat16(1.0), lb)

    qb = 256 if q % 256 == 0 else q
    pb = 1024 if p_total % 1024 == 0 else p_total
    n_q, n_p = q // qb, p_total // pb

    body = functools.partial(_snn_kernel, n_q=n_q, n_p=n_p, qb=qb, pb=pb,
                             c_real=c, p_total=p_total)
    last_q = n_q - 1
    pi_pad, aux, knn_t = pl.pallas_call(
        body,
        grid=(n_q + 1, n_p),
        in_specs=[
            pl.BlockSpec((qb, d), lambda iq, ip: (jnp.minimum(iq, last_q), 0)),
            pl.BlockSpec((d, pb), lambda iq, ip: (0, ip)),
            pl.BlockSpec((pb, cpad), lambda iq, ip: (ip, 0)),
        ],
        out_specs=[
            pl.BlockSpec((qb, cpad),
                         lambda iq, ip: (jnp.minimum(iq, last_q), 0)),
            pl.BlockSpec((8, qb), lambda iq, ip: (0, jnp.maximum(iq - 1, 0))),
            pl.BlockSpec((16, qb), lambda iq, ip: (0, jnp.maximum(iq - 1, 0))),
        ],
        out_shape=[
            jax.ShapeDtypeStruct((q, cpad), jnp.float32),
            jax.ShapeDtypeStruct((8, q), jnp.float32),
            jax.ShapeDtypeStruct((16, q), jnp.int32),
        ],
        scratch_shapes=[
            pltpu.VMEM((qb, cpad), jnp.float32),
            pltpu.VMEM((2, p_total, qb), jnp.float32),
            pltpu.VMEM((p_total, qb), jnp.float32),
        ],
    )(hb, ptb, lb)

    p_i = pi_pad[:, :c]
    density = aux[0]
    knn_indices = knn_t[:TOPK].T
    return p_i, density, knn_indices


# speedup vs baseline: 1.0613x; 1.0014x over previous
import functools

import jax
import jax.numpy as jnp
from jax.experimental import pallas as pl
from jax.experimental.pallas import tpu as pltpu

TAU = 0.07
TOPK = 10


def _topk_iteration(k, simT_ref, buf, aux_ref, knn_ref, qb, p_total):
    x = simT_ref[buf]
    iota_p = jax.lax.broadcasted_iota(jnp.int32, x.shape, 0).astype(
        jnp.float32)
    if k == 0:
        m = jnp.max(x, axis=0, keepdims=True)
    else:
        m = aux_ref[1:2, :]
    cand = jnp.where(x == m, iota_p, float(p_total))
    i = jnp.min(cand, axis=0, keepdims=True)
    if k == 0:
        aux_ref[0:1, :] = m
    else:
        aux_ref[0:1, :] += m
    knn_ref[k:k + 1, :] = i.astype(jnp.int32)
    if k < TOPK - 1:
        masked = jnp.where(iota_p == i, -jnp.inf, x)
        simT_ref[buf] = masked
        aux_ref[1:2, :] = jnp.max(masked, axis=0, keepdims=True)
    else:
        aux_ref[0:1, :] = aux_ref[0:1, :] / float(TOPK)


def _snn_kernel(hb_ref, ptb_ref, lb_ref, pi_ref, aux_ref, knn_ref,
                acc_ref, simT_ref, *, n_q, n_p, qb, pb, c_real, p_total):
    iq = pl.program_id(0)
    ip = pl.program_id(1)

    @pl.when(iq < n_q)
    def _compute():
        sim = jnp.dot(hb_ref[...], ptb_ref[...],
                      preferred_element_type=jnp.float32)
        simT_ref[iq % 2, pl.ds(ip * pb, pb), :] = sim.T

        e = jnp.exp(sim * (1.0 / TAU))
        contrib = jnp.dot(e.astype(jnp.bfloat16), lb_ref[...],
                          preferred_element_type=jnp.float32)

        @pl.when(ip == 0)
        def _init():
            acc_ref[...] = contrib

        @pl.when(ip > 0)
        def _accum():
            acc_ref[...] += contrib

        @pl.when(ip == n_p - 1)
        def _finalize_pi():
            acc = acc_ref[...]
            cpad = acc.shape[1]
            iota_c = jax.lax.broadcasted_iota(jnp.int32, (qb, cpad), 1)
            den = jnp.sum(jnp.where(iota_c == c_real, acc, 0.0), axis=1,
                          keepdims=True)
            pi_ref[...] = acc / den

    per_step = -(-TOPK // n_p)
    prev_buf = (iq + 1) % 2

    @pl.when(iq > 0)
    def _topk():
        for c in range((TOPK + per_step - 1) // per_step):
            ks = [k for k in range(c * per_step, min((c + 1) * per_step, TOPK))]

            @pl.when(ip == c)
            def _run(ks=ks):
                for k in ks:
                    _topk_iteration(k, simT_ref, prev_buf, aux_ref,
                                    knn_ref, qb, p_total)


def kernel(h, peaks, labels):
    q, d = h.shape
    p_total, c = labels.shape[0], labels.shape[1]

    h_n = h / jnp.linalg.norm(h, axis=-1, keepdims=True)
    p_n = peaks / jnp.linalg.norm(peaks, axis=-1, keepdims=True)

    hb = h_n.astype(jnp.bfloat16)
    ptb = p_n.astype(jnp.bfloat16).T

    cpad = ((c + 1 + 127) // 128) * 128
    lb = jnp.pad(labels.astype(jnp.bfloat16), ((0, 0), (0, cpad - c)))
    ones_col = (jax.lax.broadcasted_iota(jnp.int32, (1, cpad), 1) == c)
    lb = jnp.where(ones_col, jnp.bfloat16(1.0), lb)

    qb = 256 if q % 256 == 0 else q
    pb = 1024 if p_total % 1024 == 0 else p_total
    n_q, n_p = q // qb, p_total // pb

    body = functools.partial(_snn_kernel, n_q=n_q, n_p=n_p, qb=qb, pb=pb,
                             c_real=c, p_total=p_total)
    last_q = n_q - 1
    pi_pad, aux, knn_t = pl.pallas_call(
        body,
        grid=(n_q + 1, n_p),
        in_specs=[
            pl.BlockSpec((qb, d), lambda iq, ip: (jnp.minimum(iq, last_q), 0)),
            pl.BlockSpec((d, pb), lambda iq, ip: (0, ip)),
            pl.BlockSpec((pb, cpad), lambda iq, ip: (ip, 0)),
        ],
        out_specs=[
            pl.BlockSpec((qb, cpad),
                         lambda iq, ip: (jnp.minimum(iq, last_q), 0)),
            pl.BlockSpec((8, qb), lambda iq, ip: (0, jnp.maximum(iq - 1, 0))),
            pl.BlockSpec((16, qb), lambda iq, ip: (0, jnp.maximum(iq - 1, 0))),
        ],
        out_shape=[
            jax.ShapeDtypeStruct((q, cpad), jnp.float32),
            jax.ShapeDtypeStruct((8, q), jnp.float32),
            jax.ShapeDtypeStruct((16, q), jnp.int32),
        ],
        scratch_shapes=[
            pltpu.VMEM((qb, cpad), jnp.float32),
            pltpu.VMEM((2, p_total, qb), jnp.float32),
        ],
    )(hb, ptb, lb)

    p_i = pi_pad[:, :c]
    density = aux[0]
    knn_indices = knn_t[:TOPK].T
    return p_i, density, knn_indices
